# Initial kernel scaffold; baseline (speedup 1.0000x reference)
import jax
import jax.numpy as jnp
from jax.experimental import pallas as pl


def kernel(X, edge_index, edge_weight, edge_attr, Ws1, bs1, Ws2, bs2, Ws3, bs3, Wt0, Wt1, Wt2, Wt3, Wt4, Wt5):
    raise NotImplementedError("write your pallas kernel here")


# speedup vs baseline: 20.0343x; 20.0343x over previous
import jax
import jax.numpy as jnp
import numpy as np
from jax.experimental import pallas as pl
from jax.experimental.pallas import tpu as pltpu

_N = 10000
_E = 160000
_H = 128
_R = 32
_CUT = 5.0

_BN = 1000
_BE = 2000
_BEM = 2000


def _silu(x):
    return x / (1.0 + jnp.exp(-x))


def _edge_mlp_body(attr_ref, ew_ref, ws1, b1, ws2, b2,
                   w3a, b3a, w3b, b3b, w3c, b3c, out_ref):
    x = attr_ref[...]
    h1 = _silu(jnp.dot(x, ws1[...], preferred_element_type=jnp.float32) + b1[...])
    h2 = _silu(jnp.dot(h1, ws2[...], preferred_element_type=jnp.float32) + b2[...])
    r = ew_ref[...]
    c = 0.5 * (jnp.cos(r * (np.pi / _CUT)) + 1.0) * (r < _CUT).astype(jnp.float32)
    for ci, (w, b) in enumerate(((w3a, b3a), (w3b, b3b), (w3c, b3c))):
        out_ref[ci] = _silu(
            jnp.dot(h2, w[...], preferred_element_type=jnp.float32) + b[...]) * c


def _prep_body(x_ref, wt0, wt1, wt2, xn_ref, va_ref, vb_ref):
    x = x_ref[...]
    nrm = (x * x).sum(axis=0)
    xn = x / (nrm + 1.0)
    xn_ref[...] = xn
    iv = (xn[0] + xn[4] + xn[8]) * (1.0 / 3.0)
    a01 = 0.5 * (xn[1] - xn[3])
    a02 = 0.5 * (xn[2] - xn[6])
    a12 = 0.5 * (xn[5] - xn[7])
    s00 = xn[0] - iv
    s11 = xn[4] - iv
    s01 = 0.5 * (xn[1] + xn[3])
    s02 = 0.5 * (xn[2] + xn[6])
    s12 = 0.5 * (xn[5] + xn[7])
    w0 = wt0[...]
    w1 = wt1[...]
    w2 = wt2[...]
    dot = lambda a, w: jnp.dot(a, w, preferred_element_type=jnp.float32)
    va_ref[0] = dot(iv, w0)
    va_ref[1] = dot(a01, w1)
    va_ref[2] = dot(a02, w1)
    va_ref[3] = dot(a12, w1)
    vb_ref[0] = dot(s00, w2)
    vb_ref[1] = dot(s01, w2)
    vb_ref[2] = dot(s02, w2)
    vb_ref[3] = dot(s11, w2)
    vb_ref[4] = dot(s12, w2)


def _make_msg_body(comp_of_k):
    nk = len(comp_of_k)

    def _msg_body(src_ref, dst_ref, ea_ref, v_ref, msg_ref):
        @pl.when(pl.program_id(0) == 0)
        def _():
            msg_ref[...] = jnp.zeros_like(msg_ref)

        def body(i, carry):
            s = src_ref[0, 0, i]
            d = dst_ref[0, 0, i]
            es = [ea_ref[c, pl.ds(i, 1), :] for c in sorted(set(comp_of_k))]
            for k in range(nk):
                msg_ref[k, pl.ds(d, 1), :] += (
                    v_ref[k, pl.ds(s, 1), :] * es[comp_of_k[k]])
            return carry

        jax.lax.fori_loop(0, _BEM, body, 0)

    return _msg_body


def _full9(t):
    iv, a01, a02, a12, s00, s01, s02, s11, s12 = t
    return (iv + s00, s01 + a01, s02 + a02,
            s01 - a01, iv + s11, s12 + a12,
            s02 - a02, s12 - a12, iv - s00 - s11)


def _post_body(xn_ref, va_ref, vb_ref, ma_ref, mb_ref, wt3, wt4, wt5, out_ref):
    va = va_ref[...]
    vb = vb_ref[...]
    ma = ma_ref[...]
    mb = mb_ref[...]
    Y = _full9(tuple(va[k] for k in range(4)) + tuple(vb[k] for k in range(5)))
    M = _full9(tuple(ma[k] for k in range(4)) + tuple(mb[k] for k in range(5)))
    y = [[Y[0], Y[1], Y[2]], [Y[3], Y[4], Y[5]], [Y[6], Y[7], Y[8]]]
    m = [[M[0], M[1], M[2]], [M[3], M[4], M[5]], [M[6], M[7], M[8]]]
    c = [[None] * 3 for _ in range(3)]
    for i in range(3):
        for j in range(3):
            acc = m[i][0] * y[0][j] + y[i][0] * m[0][j]
            for kk in (1, 2):
                acc = acc + m[i][kk] * y[kk][j] + y[i][kk] * m[kk][j]
            c[i][j] = acc
    nrm = None
    for i in range(3):
        for j in range(3):
            t = c[i][j] * c[i][j]
            nrm = t if nrm is None else nrm + t
    inv = 1.0 / (nrm + 1.0)
    ivc = (c[0][0] + c[1][1] + c[2][2]) * (1.0 / 3.0)
    a01c = 0.5 * (c[0][1] - c[1][0])
    a02c = 0.5 * (c[0][2] - c[2][0])
    a12c = 0.5 * (c[1][2] - c[2][1])
    s00c = c[0][0] - ivc
    s11c = c[1][1] - ivc
    s01c = 0.5 * (c[0][1] + c[1][0])
    s02c = 0.5 * (c[0][2] + c[2][0])
    s12c = 0.5 * (c[1][2] + c[2][1])
    w3 = wt3[...]
    w4 = wt4[...]
    w5 = wt5[...]
    dot = lambda a, w: jnp.dot(a * inv, w, preferred_element_type=jnp.float32)
    D = _full9((dot(ivc, w3),
                dot(a01c, w4), dot(a02c, w4), dot(a12c, w4),
                dot(s00c, w5), dot(s01c, w5), dot(s02c, w5),
                dot(s11c, w5), dot(s12c, w5)))
    d = [[D[0], D[1], D[2]], [D[3], D[4], D[5]], [D[6], D[7], D[8]]]
    xn = xn_ref[...]
    for i in range(3):
        for j in range(3):
            acc = d[i][j]
            for kk in range(3):
                acc = acc + d[i][kk] * d[kk][j]
            out_ref[i * 3 + j] = xn[i * 3 + j] + acc


def kernel(X, edge_index, edge_weight, edge_attr,
           Ws1, bs1, Ws2, bs2, Ws3, bs3, Wt0, Wt1, Wt2, Wt3, Wt4, Wt5):
    f32 = jnp.float32
    X9 = X.reshape(_N, _H, 9).transpose(2, 0, 1)
    ei = edge_index.astype(jnp.int32)
    nblk = _E // _BEM
    src_b = ei[0].reshape(nblk, 1, _BEM)
    dst_b = ei[1].reshape(nblk, 1, _BEM)
    ew2 = edge_weight.reshape(_E, 1)
    b1 = bs1.reshape(1, _H)
    b2 = bs2.reshape(1, 2 * _H)
    w3s = [Ws3[:, ci::3] for ci in range(3)]
    b3s = [bs3[ci::3].reshape(1, _H) for ci in range(3)]

    full = lambda *shape: pl.BlockSpec(shape, lambda g: (0,) * len(shape))

    EA = pl.pallas_call(
        _edge_mlp_body,
        grid=(_E // _BE,),
        in_specs=[
            pl.BlockSpec((_BE, _R), lambda g: (g, 0)),
            pl.BlockSpec((_BE, 1), lambda g: (g, 0)),
            full(_R, _H), full(1, _H),
            full(_H, 2 * _H), full(1, 2 * _H),
            full(2 * _H, _H), full(1, _H),
            full(2 * _H, _H), full(1, _H),
            full(2 * _H, _H), full(1, _H),
        ],
        out_specs=pl.BlockSpec((3, _BE, _H), lambda g: (0, g, 0)),
        out_shape=jax.ShapeDtypeStruct((3, _E, _H), f32),
    )(edge_attr, ew2, Ws1, b1, Ws2, b2,
      w3s[0], b3s[0], w3s[1], b3s[1], w3s[2], b3s[2])

    Xn9, VA, VB = pl.pallas_call(
        _prep_body,
        grid=(_N // _BN,),
        in_specs=[
            pl.BlockSpec((9, _BN, _H), lambda g: (0, g, 0)),
            full(_H, _H), full(_H, _H), full(_H, _H),
        ],
        out_specs=[
            pl.BlockSpec((9, _BN, _H), lambda g: (0, g, 0)),
            pl.BlockSpec((4, _BN, _H), lambda g: (0, g, 0)),
            pl.BlockSpec((5, _BN, _H), lambda g: (0, g, 0)),
        ],
        out_shape=[
            jax.ShapeDtypeStruct((9, _N, _H), f32),
            jax.ShapeDtypeStruct((4, _N, _H), f32),
            jax.ShapeDtypeStruct((5, _N, _H), f32),
        ],
    )(X9, Wt0, Wt1, Wt2)

    def _msg_pass(nk, ea_lo, ea_n, comp_of_k, V):
        return pl.pallas_call(
            _make_msg_body(comp_of_k),
            grid=(nblk,),
            in_specs=[
                pl.BlockSpec((1, 1, _BEM), lambda g: (g, 0, 0),
                             memory_space=pltpu.SMEM),
                pl.BlockSpec((1, 1, _BEM), lambda g: (g, 0, 0),
                             memory_space=pltpu.SMEM),
                pl.BlockSpec((ea_n, _BEM, _H), lambda g: (ea_lo, g, 0)),
                pl.BlockSpec((nk, _N, _H), lambda g: (0, 0, 0)),
            ],
            out_specs=pl.BlockSpec((nk, _N, _H), lambda g: (0, 0, 0)),
            out_shape=jax.ShapeDtypeStruct((nk, _N, _H), f32),
        )(src_b, dst_b, EA, V)

    MA = _msg_pass(4, 0, 2, (0, 1, 1, 1), VA)
    MB = _msg_pass(5, 2, 1, (0, 0, 0, 0, 0), VB)

    OUT9 = pl.pallas_call(
        _post_body,
        grid=(_N // _BN,),
        in_specs=[
            pl.BlockSpec((9, _BN, _H), lambda g: (0, g, 0)),
            pl.BlockSpec((4, _BN, _H), lambda g: (0, g, 0)),
            pl.BlockSpec((5, _BN, _H), lambda g: (0, g, 0)),
            pl.BlockSpec((4, _BN, _H), lambda g: (0, g, 0)),
            pl.BlockSpec((5, _BN, _H), lambda g: (0, g, 0)),
            full(_H, _H), full(_H, _H), full(_H, _H),
        ],
        out_specs=pl.BlockSpec((9, _BN, _H), lambda g: (0, g, 0)),
        out_shape=jax.ShapeDtypeStruct((9, _N, _H), f32),
    )(Xn9, VA, VB, MA, MB, Wt3, Wt4, Wt5)

    return OUT9.transpose(1, 2, 0).reshape(_N, _H, 3, 3)
